# Initial kernel scaffold; baseline (speedup 1.0000x reference)
import jax
import jax.numpy as jnp
from jax.experimental import pallas as pl


def kernel(x, edge_index, W1, b1, g1, be1, W2, b2, g2, be2, W3, b3):
    raise NotImplementedError("write your pallas kernel here")


# speedup vs baseline: 11.2062x; 11.2062x over previous
import functools
import math

import jax
import jax.numpy as jnp
from jax import lax
from jax.experimental import pallas as pl
from jax.experimental.pallas import tpu as pltpu
from jax.experimental.pallas import tpu_sc as plsc

N = 10000
E = 320000
D = 128
H = 128
O = 64
EPS = 1e-5

NC = 2
NS = 16
NW = NC * NS
EPW = E // NW
CH = 80
NCHUNK = EPW // CH
NP = 10240
RPT = NP // NS
R = 1024


def _sc_mesh():
    return plsc.VectorSubcoreMesh(core_axis_name="c", subcore_axis_name="s",
                                  num_cores=NC, num_subcores=NS)


def _hist_body(dst_hbm, zcol_hbm, ones_hbm, out_hbm, idx_v, ones_v, hist_sh, sem):
    c = lax.axis_index("c")
    s = lax.axis_index("s")
    wid = c * NS + s
    pltpu.sync_copy(ones_hbm, ones_v)
    pltpu.sync_copy(zcol_hbm.at[pl.ds(s * RPT, RPT)],
                    hist_sh.at[pl.ds(s * RPT, RPT)])
    plsc.subcore_barrier()

    def step(j, carry):
        base = pl.multiple_of(wid * EPW + j * CH, 8)
        pltpu.sync_copy(dst_hbm.at[pl.ds(base, CH)], idx_v)
        pltpu.sync_copy(ones_v, hist_sh.at[idx_v], add=True)
        return carry

    lax.fori_loop(0, NCHUNK, step, 0)
    plsc.subcore_barrier()
    pltpu.sync_copy(hist_sh.at[pl.ds(s * RPT, RPT)],
                    out_hbm.at[c, pl.ds(s * RPT, RPT)])


def _hist_call(dst, zcol, ones_col):
    return pl.kernel(
        _hist_body,
        out_type=jax.ShapeDtypeStruct((NC, NP), jnp.float32),
        mesh=_sc_mesh(),
        scratch_types=[
            pltpu.VMEM((CH,), jnp.int32),
            pltpu.VMEM((CH,), jnp.float32),
            pltpu.VMEM_SHARED((NP,), jnp.float32),
            pltpu.SemaphoreType.DMA,
        ],
    )(dst, zcol, ones_col)


def _agg_body(hs_hbm, src_hbm, dst_hbm, z_hbm, out_hbm,
              sidx_v, didx_v, rows_v, acc_sh, sem):
    c = lax.axis_index("c")
    s = lax.axis_index("s")
    wid = c * NS + s
    pltpu.sync_copy(z_hbm.at[pl.ds(s * RPT, RPT)],
                    acc_sh.at[pl.ds(s * RPT, RPT)])
    plsc.subcore_barrier()

    def step(j, carry):
        base = pl.multiple_of(wid * EPW + j * CH, 8)
        pltpu.sync_copy(src_hbm.at[pl.ds(base, CH)], sidx_v)
        pltpu.async_copy(hs_hbm.at[sidx_v], rows_v, sem).wait()
        pltpu.sync_copy(dst_hbm.at[pl.ds(base, CH)], didx_v)
        pltpu.sync_copy(rows_v, acc_sh.at[didx_v], add=True)
        return carry

    lax.fori_loop(0, NCHUNK, step, 0)
    plsc.subcore_barrier()
    pltpu.sync_copy(acc_sh.at[pl.ds(s * RPT, RPT)],
                    out_hbm.at[c, pl.ds(s * RPT, RPT)])


def _agg_call(hs, src, dst, zpad, width):
    return pl.kernel(
        _agg_body,
        out_type=jax.ShapeDtypeStruct((NC, NP, width), jnp.float32),
        mesh=_sc_mesh(),
        scratch_types=[
            pltpu.VMEM((CH,), jnp.int32),
            pltpu.VMEM((CH,), jnp.int32),
            pltpu.VMEM((CH, width), jnp.float32),
            pltpu.VMEM_SHARED((NP, width), jnp.float32),
            pltpu.SemaphoreType.DMA,
        ],
    )(hs, src, dst, zpad)


def _layer1_body(x_ref, w_ref, degp_ref, hs_ref, dinv_ref):
    deg_row = degp_ref[0] + degp_ref[1]
    deg = lax.dot_general(deg_row, jnp.ones((1, 1), jnp.float32),
                          (((0,), (0,)), ((), ()))) + 1.0
    dv = lax.rsqrt(deg)
    h = jnp.dot(x_ref[...], w_ref[...], preferred_element_type=jnp.float32)
    hs_ref[...] = h * dv
    dinv_ref[...] = dv


def _layer1_call(x, W1, degp):
    grid = (pl.cdiv(N, R),)
    return pl.pallas_call(
        _layer1_body,
        grid=grid,
        in_specs=[
            pl.BlockSpec((R, D), lambda i: (i, 0)),
            pl.BlockSpec((D, H), lambda i: (0, 0)),
            pl.BlockSpec((NC, 1, R), lambda i: (0, 0, i)),
        ],
        out_specs=[
            pl.BlockSpec((R, H), lambda i: (i, 0)),
            pl.BlockSpec((R, 1), lambda i: (i, 0)),
        ],
        out_shape=[
            jax.ShapeDtypeStruct((N, H), jnp.float32),
            jax.ShapeDtypeStruct((N, 1), jnp.float32),
        ],
    )(x, W1, degp)


_BN_SCALE = 1.0 / math.sqrt(1.0 + EPS)


def _mid_body(p_ref, hsp_ref, dinv_ref, b_ref, g_ref, be_ref, w_ref, out_ref):
    dv = dinv_ref[...]
    agg = p_ref[0] + p_ref[1] + hsp_ref[...]
    o = agg * dv + b_ref[...]
    y = jnp.maximum(o * (g_ref[...] * _BN_SCALE) + be_ref[...], 0.0)
    out_ref[...] = jnp.dot(y, w_ref[...], preferred_element_type=jnp.float32) * dv


def _mid_call(p, hs_prev, dinv, b, g, be, Wn, win, wout):
    grid = (pl.cdiv(N, R),)
    return pl.pallas_call(
        _mid_body,
        grid=grid,
        in_specs=[
            pl.BlockSpec((NC, R, win), lambda i: (0, i, 0)),
            pl.BlockSpec((R, win), lambda i: (i, 0)),
            pl.BlockSpec((R, 1), lambda i: (i, 0)),
            pl.BlockSpec((1, win), lambda i: (0, 0)),
            pl.BlockSpec((1, win), lambda i: (0, 0)),
            pl.BlockSpec((1, win), lambda i: (0, 0)),
            pl.BlockSpec((win, wout), lambda i: (0, 0)),
        ],
        out_specs=pl.BlockSpec((R, wout), lambda i: (i, 0)),
        out_shape=jax.ShapeDtypeStruct((N, wout), jnp.float32),
    )(p, hs_prev, dinv, b.reshape(1, win), g.reshape(1, win),
      be.reshape(1, win), Wn)


def _final_body(p_ref, hsp_ref, dinv_ref, b_ref, out_ref):
    agg = (p_ref[0] + p_ref[1] + hsp_ref[...])[:, :O]
    o = agg * dinv_ref[...] + b_ref[...]
    out_ref[...] = jax.nn.sigmoid(o)


def _final_call(p, hs_prev, dinv, b):
    grid = (pl.cdiv(N, R),)
    return pl.pallas_call(
        _final_body,
        grid=grid,
        in_specs=[
            pl.BlockSpec((NC, R, H), lambda i: (0, i, 0)),
            pl.BlockSpec((R, H), lambda i: (i, 0)),
            pl.BlockSpec((R, 1), lambda i: (i, 0)),
            pl.BlockSpec((1, O), lambda i: (0, 0)),
        ],
        out_specs=pl.BlockSpec((R, O), lambda i: (i, 0)),
        out_shape=jax.ShapeDtypeStruct((N, O), jnp.float32),
    )(p, hs_prev, dinv, b.reshape(1, O))


def kernel(x, edge_index, W1, b1, g1, be1, W2, b2, g2, be2, W3, b3):
    src = edge_index[0].astype(jnp.int32)
    dst = edge_index[1].astype(jnp.int32)

    zcol = jnp.zeros((NP,), jnp.float32)
    ones_col = jnp.ones((CH,), jnp.float32)
    z_h = jnp.zeros((NP, H), jnp.float32)
    W3p = jnp.concatenate([W3, jnp.zeros((H, H - O), jnp.float32)], axis=1)

    degp = _hist_call(dst, zcol, ones_col)
    hs1, dinv = _layer1_call(x, W1, degp.reshape(NC, 1, NP))

    p1 = _agg_call(hs1, src, dst, z_h, H)
    hs2 = _mid_call(p1, hs1, dinv, b1, g1, be1, W2, H, H)

    p2 = _agg_call(hs2, src, dst, z_h, H)
    hs3 = _mid_call(p2, hs2, dinv, b2, g2, be2, W3p, H, H)

    p3 = _agg_call(hs3, src, dst, z_h, H)
    return _final_call(p3, hs3, dinv, b3)
